# baseline (speedup 1.0000x reference)
import numpy as np
import jax
import jax.numpy as jnp
from jax.experimental import pallas as pl
import jax.experimental.pallas.tpu as pltpu

NLAT, NLON, CH = 73, 144, 256
NPTS = NLAT * NLON
MB = 11
NEG = np.float32(-1e30)
PN = 2 * NLON
PK = 3 * NLON

SB = 2304
NSB = 5
CW = 384
NCK = 8
KCHUNKS = (NPTS + CW - 1) // CW
WSUB = 128
WWIN = 448
NSUB = SB // WSUB
G = NSB * NSUB
SHIFT = 32
SW = NCK * CW + 128


def _nbr_inclusion():
    lats = np.pi * np.arange(NLAT) / (NLAT - 1)
    lons = np.linspace(0.0, 2.0 * np.pi, NLON + 1)[:-1]
    cutoff = np.pi / (NLAT - 1)
    per_row = []
    for t in range(NLAT):
        alpha = -lats[t]
        beta = lons[None, :]
        gamma = lats[:, None]
        z = -np.cos(beta) * np.sin(alpha) * np.sin(gamma) + np.cos(alpha) * np.cos(gamma)
        x = np.cos(alpha) * np.cos(beta) * np.sin(gamma) + np.cos(gamma) * np.sin(alpha)
        y = np.sin(beta) * np.sin(gamma)
        norm = np.sqrt(x * x + y * y + z * z)
        th = np.arccos(np.clip(z / norm, -1.0, 1.0))
        per_row.append(th <= cutoff)
    return per_row


def _build_masks():
    inc = _nbr_inclusion()
    wm = np.zeros((G, WSUB, WWIN), np.int8)
    for g in range(G):
        w0 = WSUB * g
        for t in range(WSUB):
            w = w0 + t
            if w >= NPTS:
                continue
            l, lw = divmod(w, NLON)
            if l < 2 or l > NLAT - 3:
                continue
            base = w0 - 160
            for d in range(-MB, MB + 1):
                if inc[l][l, d % NLON]:
                    j = l * NLON + (lw + d) % NLON
                    wm[g, t, j - base] = 1
            if inc[l][l - 1, 0]:
                wm[g, t, (w - NLON) - base] = 1
            if inc[l][l + 1, 0]:
                wm[g, t, (w + NLON) - base] = 1
    w = np.arange(NLON)
    doff = (w[None, :] - w[:, None]) % NLON
    pm = np.zeros((2, PN, PK), np.float32)
    for pi, p in enumerate((0, 1)):
        for ri, r in enumerate((0, 1, 2)):
            pm[0, pi * NLON:(pi + 1) * NLON, ri * NLON:(ri + 1) * NLON] = inc[p][r][doff]
    for pi, p in enumerate((NLAT - 2, NLAT - 1)):
        for ri, r in enumerate((NLAT - 3, NLAT - 2, NLAT - 1)):
            pm[1, pi * NLON:(pi + 1) * NLON, ri * NLON:(ri + 1) * NLON] = inc[p][r][doff]
    return wm, pm


_WM_NP, _PM_NP = _build_masks()


def _dot(a, b, dims):
    return jax.lax.dot_general(a, b, (dims, ((), ())),
                               preferred_element_type=jnp.float32)


def _polar(qslab, kslab, vslab, pm):
    logits = _dot(qslab, kslab, ((0,), (0,)))
    lm = jnp.where(pm > 0, logits, NEG)
    mx = jnp.max(lm, axis=1, keepdims=True)
    a = jnp.exp(lm - mx) * pm
    den = jnp.sum(a, axis=1, keepdims=True)
    return _dot(vslab, a / den, ((1,), (1,)))


def _main_body(*refs):
    xq_ref = refs[0]
    k_refs = refs[1:1 + NCK]
    v_refs = refs[1 + NCK:1 + 2 * NCK]
    qw_refs = refs[1 + 2 * NCK:1 + 3 * NCK]
    ws_ref, bs_ref, wm_ref, pm_ref, o_ref, kvp_ref = refs[1 + 3 * NCK:]
    i = pl.program_id(0)

    qp = _dot(ws_ref[0], xq_ref[...], ((1,), (0,))) + bs_ref[0]
    for j in range(NCK):
        kvp_ref[0, :, SHIFT + j * CW:SHIFT + (j + 1) * CW] = (
            _dot(ws_ref[1], k_refs[j][...], ((1,), (0,))) + bs_ref[1])
        kvp_ref[1, :, SHIFT + j * CW:SHIFT + (j + 1) * CW] = (
            _dot(ws_ref[2], v_refs[j][...], ((1,), (0,))) + bs_ref[2])
    qwj = jnp.concatenate([r[...] for r in qw_refs], axis=1)

    outs = []
    for s in range(NSUB):
        qb = qp[:, s * WSUB:(s + 1) * WSUB]
        j0 = s * WSUB + 224
        kw = kvp_ref[0, :, SHIFT + j0:SHIFT + j0 + WWIN]
        vw = kvp_ref[1, :, SHIFT + j0:SHIFT + j0 + WWIN]
        wmask = wm_ref[s].astype(jnp.float32) * qwj[:, j0:j0 + WWIN]
        logits = _dot(qb, kw, ((0,), (0,)))
        lm = jnp.where(wmask > 0, logits, NEG)
        mx = jnp.max(lm, axis=1, keepdims=True)
        a = jnp.exp(lm - mx) * wmask
        den = jnp.maximum(jnp.sum(a, axis=1, keepdims=True), np.float32(1e-30))
        outs.append(_dot(vw, a / den, ((1,), (1,))))
    o_ref[...] = jnp.concatenate(outs, axis=1).reshape(CH, SB // NLON, NLON)

    @pl.when(i == 0)
    def _north():
        on = _polar(qp[:, 0:PN], kvp_ref[0, :, 416:848],
                    kvp_ref[1, :, 416:848], pm_ref[0])
        o_ref[:, 0:2, :] = on.reshape(CH, 2, NLON)

    @pl.when(i == NSB - 1)
    def _south():
        os_ = _polar(qp[:, 1008:1008 + PN], kvp_ref[0, :, 1280:1712],
                     kvp_ref[1, :, 1280:1712], pm_ref[1])
        o_ref[:, 7:9, :] = os_.reshape(CH, 2, NLON)


def kernel(qo, ki, vi, q_weights, k_weights, v_weights, q_bias, k_bias, v_bias,
           quad_weights, psi_col_idx, psi_roff_idx):
    del psi_col_idx, psi_roff_idx
    qf = qo[0].reshape(CH, NPTS)
    kf = ki[0].reshape(CH, NPTS)
    vf = vi[0].reshape(CH, NPTS)
    ws = jnp.stack([q_weights, k_weights, v_weights])
    bs = jnp.stack([q_bias, k_bias, v_bias]).reshape(3, CH, 1)
    qw = quad_weights.astype(jnp.float32)
    qwj = jnp.repeat(qw, NLON).reshape(1, NPTS)
    pmw = jnp.asarray(_PM_NP) * jnp.stack(
        [jnp.repeat(qw[0:3], NLON), jnp.repeat(qw[NLAT - 3:], NLON)])[:, None, :]
    wm = jnp.asarray(_WM_NP)

    def cmap(j):
        return lambda i, jj=j: (0, jnp.clip(6 * i - 1 + jj, 0, KCHUNKS - 1))

    kv_specs = [pl.BlockSpec((CH, CW), cmap(j)) for j in range(NCK)]
    qw_specs = [pl.BlockSpec((1, CW), cmap(j)) for j in range(NCK)]
    full = lambda s: pl.BlockSpec(s, lambda i: tuple(0 for _ in s))
    out = pl.pallas_call(
        _main_body,
        grid=(NSB,),
        in_specs=[pl.BlockSpec((CH, SB), lambda i: (0, i))]
                 + kv_specs + kv_specs + qw_specs
                 + [full((3, CH, CH)), full((3, CH, 1)),
                    pl.BlockSpec((NSUB, WSUB, WWIN), lambda i: (i, 0, 0)),
                    full((2, PN, PK))],
        out_specs=pl.BlockSpec((CH, SB // NLON, NLON), lambda i: (0, i, 0)),
        out_shape=jax.ShapeDtypeStruct((CH, NLAT, NLON), jnp.float32),
        scratch_shapes=[pltpu.VMEM((2, CH, SW), jnp.float32)],
    )(qf, *([kf] * NCK), *([vf] * NCK), *([qwj] * NCK), ws, bs, wm, pmw)

    return out.reshape(1, CH, NLAT, NLON)


# speedup vs baseline: 1.1387x; 1.1387x over previous
import numpy as np
import jax
import jax.numpy as jnp
from jax.experimental import pallas as pl
import jax.experimental.pallas.tpu as pltpu

NLAT, NLON, CH = 73, 144, 256
NPTS = NLAT * NLON
MB = 11
NEG = np.float32(-1e30)
PN = 2 * NLON
PK = 3 * NLON

SB = 3072
NSB = 4
CW = 512
NCK = 8
KCHUNKS = (NPTS + CW - 1) // CW
WSUB = 128
WWIN = 448
NSUB = SB // WSUB
G = NSB * NSUB
SHIFT = 32
SW = NCK * CW + 128


def _nbr_inclusion():
    lats = np.pi * np.arange(NLAT) / (NLAT - 1)
    lons = np.linspace(0.0, 2.0 * np.pi, NLON + 1)[:-1]
    cutoff = np.pi / (NLAT - 1)
    per_row = []
    for t in range(NLAT):
        alpha = -lats[t]
        beta = lons[None, :]
        gamma = lats[:, None]
        z = -np.cos(beta) * np.sin(alpha) * np.sin(gamma) + np.cos(alpha) * np.cos(gamma)
        x = np.cos(alpha) * np.cos(beta) * np.sin(gamma) + np.cos(gamma) * np.sin(alpha)
        y = np.sin(beta) * np.sin(gamma)
        norm = np.sqrt(x * x + y * y + z * z)
        th = np.arccos(np.clip(z / norm, -1.0, 1.0))
        per_row.append(th <= cutoff)
    return per_row


def _build_masks():
    inc = _nbr_inclusion()
    wm = np.zeros((G, WSUB, WWIN), np.int8)
    for g in range(G):
        w0 = WSUB * g
        for t in range(WSUB):
            w = w0 + t
            if w >= NPTS:
                continue
            l, lw = divmod(w, NLON)
            if l < 2 or l > NLAT - 3:
                continue
            base = w0 - 160
            for d in range(-MB, MB + 1):
                if inc[l][l, d % NLON]:
                    j = l * NLON + (lw + d) % NLON
                    wm[g, t, j - base] = 1
            if inc[l][l - 1, 0]:
                wm[g, t, (w - NLON) - base] = 1
            if inc[l][l + 1, 0]:
                wm[g, t, (w + NLON) - base] = 1
    w = np.arange(NLON)
    doff = (w[None, :] - w[:, None]) % NLON
    pm = np.zeros((2, PN, PK), np.float32)
    for pi, p in enumerate((0, 1)):
        for ri, r in enumerate((0, 1, 2)):
            pm[0, pi * NLON:(pi + 1) * NLON, ri * NLON:(ri + 1) * NLON] = inc[p][r][doff]
    for pi, p in enumerate((NLAT - 2, NLAT - 1)):
        for ri, r in enumerate((NLAT - 3, NLAT - 2, NLAT - 1)):
            pm[1, pi * NLON:(pi + 1) * NLON, ri * NLON:(ri + 1) * NLON] = inc[p][r][doff]
    return wm, pm


_WM_NP, _PM_NP = _build_masks()


def _dot(a, b, dims):
    return jax.lax.dot_general(a, b, (dims, ((), ())),
                               preferred_element_type=jnp.float32)


def _polar(qslab, kslab, vslab, pm):
    logits = _dot(qslab, kslab, ((0,), (0,)))
    lm = jnp.where(pm > 0, logits, NEG)
    mx = jnp.max(lm, axis=1, keepdims=True)
    a = jnp.exp(lm - mx) * pm
    den = jnp.sum(a, axis=1, keepdims=True)
    return _dot(vslab, a / den, ((1,), (1,)))


def _main_body(*refs):
    xq_ref = refs[0]
    k_refs = refs[1:1 + NCK]
    v_refs = refs[1 + NCK:1 + 2 * NCK]
    qw_refs = refs[1 + 2 * NCK:1 + 3 * NCK]
    ws_ref, bs_ref, wm_ref, pm_ref, o_ref, kvp_ref = refs[1 + 3 * NCK:]
    i = pl.program_id(0)

    qp = _dot(ws_ref[0], xq_ref[...], ((1,), (0,))) + bs_ref[0]
    for j in range(NCK):
        kvp_ref[0, :, SHIFT + j * CW:SHIFT + (j + 1) * CW] = (
            _dot(ws_ref[1], k_refs[j][...], ((1,), (0,))) + bs_ref[1])
        kvp_ref[1, :, SHIFT + j * CW:SHIFT + (j + 1) * CW] = (
            _dot(ws_ref[2], v_refs[j][...], ((1,), (0,))) + bs_ref[2])
    qwj = jnp.concatenate([r[...] for r in qw_refs], axis=1)

    for s in range(NSUB):
        qb = qp[:, s * WSUB:(s + 1) * WSUB]
        j0 = s * WSUB + 352
        kw = kvp_ref[0, :, SHIFT + j0:SHIFT + j0 + WWIN]
        vw = kvp_ref[1, :, SHIFT + j0:SHIFT + j0 + WWIN]
        wmask = wm_ref[s].astype(jnp.float32) * qwj[:, j0:j0 + WWIN]
        logits = _dot(qb, kw, ((0,), (0,)))
        lm = jnp.where(wmask > 0, logits, NEG)
        mx = jnp.max(lm, axis=1, keepdims=True)
        a = jnp.exp(lm - mx) * wmask
        den = jnp.maximum(jnp.sum(a, axis=1, keepdims=True), np.float32(1e-30))
        o_ref[:, s * WSUB:(s + 1) * WSUB] = _dot(vw, a / den, ((1,), (1,)))

    @pl.when(i == 0)
    def _north():
        on = _polar(qp[:, 0:PN], kvp_ref[0, :, 544:976],
                    kvp_ref[1, :, 544:976], pm_ref[0])
        o_ref[:, 0:PN] = on

    @pl.when(i == NSB - 1)
    def _south():
        os_ = _polar(qp[:, 1008:1008 + PN], kvp_ref[0, :, 1408:1840],
                     kvp_ref[1, :, 1408:1840], pm_ref[1])
        o_ref[:, 1008:1008 + PN] = os_


def kernel(qo, ki, vi, q_weights, k_weights, v_weights, q_bias, k_bias, v_bias,
           quad_weights, psi_col_idx, psi_roff_idx):
    del psi_col_idx, psi_roff_idx
    qf = qo[0].reshape(CH, NPTS)
    kf = ki[0].reshape(CH, NPTS)
    vf = vi[0].reshape(CH, NPTS)
    ws = jnp.stack([q_weights, k_weights, v_weights])
    bs = jnp.stack([q_bias, k_bias, v_bias]).reshape(3, CH, 1)
    qw = quad_weights.astype(jnp.float32)
    qwj = jnp.repeat(qw, NLON).reshape(1, NPTS)
    pmw = jnp.asarray(_PM_NP) * jnp.stack(
        [jnp.repeat(qw[0:3], NLON), jnp.repeat(qw[NLAT - 3:], NLON)])[:, None, :]
    wm = jnp.asarray(_WM_NP)

    def cmap(j):
        return lambda i, jj=j: (0, jnp.clip(6 * i - 1 + jj, 0, KCHUNKS - 1))

    kv_specs = [pl.BlockSpec((CH, CW), cmap(j)) for j in range(NCK)]
    qw_specs = [pl.BlockSpec((1, CW), cmap(j)) for j in range(NCK)]
    full = lambda s: pl.BlockSpec(s, lambda i: tuple(0 for _ in s))
    out = pl.pallas_call(
        _main_body,
        grid=(NSB,),
        in_specs=[pl.BlockSpec((CH, SB), lambda i: (0, i))]
                 + kv_specs + kv_specs + qw_specs
                 + [full((3, CH, CH)), full((3, CH, 1)),
                    pl.BlockSpec((NSUB, WSUB, WWIN), lambda i: (i, 0, 0)),
                    full((2, PN, PK))],
        out_specs=pl.BlockSpec((CH, SB), lambda i: (0, i)),
        out_shape=jax.ShapeDtypeStruct((CH, NPTS), jnp.float32),
        scratch_shapes=[pltpu.VMEM((2, CH, SW), jnp.float32)],
    )(qf, *([kf] * NCK), *([vf] * NCK), *([qwj] * NCK), ws, bs, wm, pmw)

    return out.reshape(1, CH, NLAT, NLON)
